# baseline (speedup 1.0000x reference)
import functools

import jax
import jax.numpy as jnp
from jax import lax
from jax.experimental import pallas as pl
from jax.experimental.pallas import tpu as pltpu
from jax.experimental.pallas import tpu_sc as plsc

M = 100000
D = 128
DMSG = 256
B = 16384
NW = 32
BPW = B // NW
RNG = 3136
RNG_LAST = M - (NW - 1) * RNG
NVREG = B // 16
NTAB = RNG // 16
CAP = RNG + 64

_MESH = dict(core_axis_name="c", subcore_axis_name="s", num_cores=2,
             num_subcores=16)


def _wid():
    return lax.axis_index("s") * 2 + lax.axis_index("c")


def _lane_iota():
    return lax.iota(jnp.int32, 16)


def _shift_up(x):
    idx = jnp.minimum(_lane_iota() + 1, 15)
    dn = lax.GatherDimensionNumbers(
        offset_dims=(), collapsed_slice_dims=(0,), start_index_map=(0,))
    return lax.gather(x, idx[:, None], dn, (1,),
                      mode=lax.GatherScatterMode.PROMISE_IN_BOUNDS)


def _splat0(x):
    idx = jnp.zeros((16,), jnp.int32)
    dn = lax.GatherDimensionNumbers(
        offset_dims=(), collapsed_slice_dims=(0,), start_index_map=(0,))
    return lax.gather(x, idx[:, None], dn, (1,),
                      mode=lax.GatherScatterMode.PROMISE_IN_BOUNDS)


@functools.partial(
    pl.kernel,
    out_type=jax.ShapeDtypeStruct((B, D), jnp.float32),
    mesh=plsc.VectorSubcoreMesh(**_MESH),
    scratch_types=[
        pltpu.VMEM((BPW,), jnp.int32),
        pltpu.VMEM((BPW, D), jnp.float32),
        pltpu.SemaphoreType.DMA,
    ],
)
def _sc_gather(mem_hbm, ids_hbm, cur_hbm, idx_v, rows_v, sem):
    base = _wid() * BPW
    pltpu.sync_copy(ids_hbm.at[pl.ds(base, BPW)], idx_v)
    for j in range(BPW // 128):
        pltpu.async_copy(mem_hbm.at[idx_v.at[pl.ds(j * 128, 128)]],
                         rows_v.at[pl.ds(j * 128, 128)], sem)
    for j in range(BPW // 128):
        pltpu.make_async_copy(mem_hbm.at[idx_v.at[pl.ds(j * 128, 128)]],
                              rows_v.at[pl.ds(j * 128, 128)], sem).wait()
    pltpu.sync_copy(rows_v, cur_hbm.at[pl.ds(base, BPW)])


def _gru_body(msg_ref, cur_ref, wih_ref, whh_ref, bih_ref, bhh_ref,
              g_ref, bt_ref, out_ref):
    msg = msg_ref[...]
    cur = cur_ref[...]
    dn = (((1,), (1,)), ((), ()))
    gi = lax.dot_general(msg, wih_ref[...], dn,
                         preferred_element_type=jnp.float32) + bih_ref[...]
    gh = lax.dot_general(cur, whh_ref[...], dn,
                         preferred_element_type=jnp.float32) + bhh_ref[...]
    r = jax.nn.sigmoid(gi[:, :D] + gh[:, :D])
    z = jax.nn.sigmoid(gi[:, D:2 * D] + gh[:, D:2 * D])
    n = jnp.tanh(gi[:, 2 * D:] + r * gh[:, 2 * D:])
    h = (1.0 - z) * n + z * cur
    mu = jnp.mean(h, axis=-1, keepdims=True)
    var = jnp.mean((h - mu) ** 2, axis=-1, keepdims=True)
    out_ref[...] = (h - mu) * lax.rsqrt(var + 1e-5) * g_ref[...] + bt_ref[...]


_GRU_BLK = 512


def _tc_gru(cur, msgs, W_ih, W_hh, b_ih, b_hh, g, bt):
    grid = B // _GRU_BLK
    return pl.pallas_call(
        _gru_body,
        grid=(grid,),
        in_specs=[
            pl.BlockSpec((_GRU_BLK, DMSG), lambda i: (i, 0)),
            pl.BlockSpec((_GRU_BLK, D), lambda i: (i, 0)),
            pl.BlockSpec((3 * D, DMSG), lambda i: (0, 0)),
            pl.BlockSpec((3 * D, D), lambda i: (0, 0)),
            pl.BlockSpec((1, 3 * D), lambda i: (0, 0)),
            pl.BlockSpec((1, 3 * D), lambda i: (0, 0)),
            pl.BlockSpec((1, D), lambda i: (0, 0)),
            pl.BlockSpec((1, D), lambda i: (0, 0)),
        ],
        out_specs=pl.BlockSpec((_GRU_BLK, D), lambda i: (i, 0)),
        out_shape=jax.ShapeDtypeStruct((B, D), jnp.float32),
    )(msgs, cur, W_ih, W_hh, b_ih, b_hh, g, bt)


_NCB = 4
_CPR = 128


def _emit_range_copy(src, dst, base, n_chunks, tail_rows, bufs, in_sems,
                     out_sems, compute_segment):
    def _in(k):
        off = base + k * _CPR
        return pltpu.make_async_copy(src.at[pl.ds(off, _CPR)],
                                     bufs.at[k % _NCB], in_sems.at[k % _NCB])

    def _out(k):
        off = base + k * _CPR
        return pltpu.make_async_copy(bufs.at[k % _NCB],
                                     dst.at[pl.ds(off, _CPR)],
                                     out_sems.at[k % _NCB])

    waited = set()
    for k in range(min(_NCB, n_chunks)):
        _in(k).start()
    for k in range(n_chunks):
        compute_segment(k, n_chunks)
        j = k - 2
        if j >= 0 and j + _NCB < n_chunks:
            _out(j).wait()
            waited.add(j)
            _in(j + _NCB).start()
        _in(k).wait()
        _out(k).start()
    for k in range(n_chunks):
        if k not in waited:
            _out(k).wait()
    if tail_rows:
        off = base + n_chunks * _CPR
        pltpu.sync_copy(src.at[pl.ds(off, tail_rows)],
                        bufs.at[0, pl.ds(0, tail_rows)])
        pltpu.sync_copy(bufs.at[0, pl.ds(0, tail_rows)],
                        dst.at[pl.ds(off, tail_rows)])


@functools.partial(
    pl.kernel,
    out_type=(jax.ShapeDtypeStruct((M, D), jnp.float32),
              jax.ShapeDtypeStruct((M,), jnp.float32)),
    mesh=plsc.VectorSubcoreMesh(**_MESH),
    scratch_types=[
        pltpu.VMEM((B,), jnp.int32),
        pltpu.VMEM((RNG,), jnp.int32),
        pltpu.VMEM((CAP,), jnp.int32),
        pltpu.VMEM((CAP,), jnp.int32),
        pltpu.VMEM((128,), jnp.int32),
        pltpu.VMEM((128,), jnp.int32),
        pltpu.VMEM((128, D), jnp.float32),
        pltpu.VMEM((128,), jnp.float32),
        pltpu.VMEM((_NCB, _CPR, D), jnp.float32),
        pltpu.VMEM((RNG,), jnp.float32),
        pltpu.SemaphoreType.DMA((_NCB,)),
        pltpu.SemaphoreType.DMA((_NCB,)),
        pltpu.SemaphoreType.DMA,
        pltpu.SemaphoreType.DMA,
    ],
    compiler_params=pltpu.CompilerParams(needs_layout_passes=False),
)
def _sc_scatter(mem_hbm, lu_hbm, ids_hbm, ts_hbm, new_hbm,
                outm_hbm, outl_hbm,
                ids_v, tab_v, dstf_v, srcf_v, idxc_v, srcc_v, rows_v, tsb_v,
                cpb_v, lub_v, in_sems, out_sems, sem, sem2):
    wid = _wid()
    base = wid * RNG
    lane = _lane_iota()

    pltpu.sync_copy(ids_hbm, ids_v)

    def _clear(t, carry):
        tab_v[pl.ds(t * 16, 16)] = jnp.full((16,), -1, jnp.int32)
        return carry
    lax.fori_loop(0, NTAB, _clear, 0)

    _UNR = 4

    def _scan(g, carry):
        comps = []
        for u in range(_UNR):
            v = g * _UNR + u
            ids16 = ids_v[pl.ds(v * 16, 16)]
            comps.append(plsc.sort_key_val(
                ids16 * 16384 + (v * 16 + lane),
                ids16 * 16384 + (v * 16 + lane))[0])
        for comp in comps:
            sid = lax.shift_right_logical(comp, 14)
            si = lax.bitwise_and(comp, 16383)
            nxt = _shift_up(sid)
            m = ((sid >= base) & (sid < base + RNG)
                 & ((sid != nxt) | (lane == 15)))
            plsc.store_scatter(tab_v, [sid - base], si, mask=m)
        return carry

    def _scan_segment(k, n_chunks):
        ngrp = NVREG // _UNR
        lo = k * ngrp // n_chunks
        hi = (k + 1) * ngrp // n_chunks
        lax.fori_loop(lo, hi, _scan, 0)

    @pl.when(wid < NW - 1)
    def _():
        _emit_range_copy(mem_hbm, outm_hbm, base, RNG // _CPR, RNG % _CPR,
                         cpb_v, in_sems, out_sems, _scan_segment)
        pltpu.sync_copy(lu_hbm.at[pl.ds(base, RNG)], lub_v)
        pltpu.sync_copy(lub_v, outl_hbm.at[pl.ds(base, RNG)])

    @pl.when(wid == NW - 1)
    def _():
        _emit_range_copy(mem_hbm, outm_hbm, base, RNG_LAST // _CPR,
                         RNG_LAST % _CPR, cpb_v, in_sems, out_sems,
                         _scan_segment)
        pltpu.sync_copy(lu_hbm.at[pl.ds(base, RNG_LAST)],
                        lub_v.at[pl.ds(0, RNG_LAST)])
        pltpu.sync_copy(lub_v.at[pl.ds(0, RNG_LAST)],
                        outl_hbm.at[pl.ds(base, RNG_LAST)])

    def _compact(t, off):
        tv = tab_v[pl.ds(t * 16, 16)]
        m = tv >= 0
        pc = plsc.cumsum(jnp.where(m, 1, 0).astype(jnp.int32))
        tgt = off + pc - 1
        plsc.store_scatter(dstf_v, [tgt], base + t * 16 + lane, mask=m)
        plsc.store_scatter(srcf_v, [tgt], tv, mask=m)
        return off + jnp.max(plsc.all_reduce_population_count(m))
    cnt = lax.fori_loop(0, NTAB, _compact, jnp.int32(0))

    nch = (cnt + 127) // 128

    @pl.when(cnt > 0)
    def _():
        pad_end = nch * 128

        def _pad(k, carry):
            pos = cnt + k * 16 + lane
            pm = pos < pad_end
            j = jnp.minimum(pos - cnt, cnt - 1)
            dv = plsc.load_gather(dstf_v, [j])
            sv = plsc.load_gather(srcf_v, [j])
            plsc.store_scatter(dstf_v, [pos], dv, mask=pm)
            plsc.store_scatter(srcf_v, [pos], sv, mask=pm)
            return carry
        lax.fori_loop(0, 8, _pad, 0)

    def _chunk(ch, carry):
        def _fill(j, carry2):
            idxc_v[pl.ds(j * 16, 16)] = dstf_v[pl.ds(ch * 128 + j * 16, 16)]
            srcc_v[pl.ds(j * 16, 16)] = srcf_v[pl.ds(ch * 128 + j * 16, 16)]
            return carry2
        lax.fori_loop(0, 8, _fill, 0)
        pltpu.async_copy(new_hbm.at[srcc_v], rows_v, sem)
        pltpu.async_copy(ts_hbm.at[srcc_v], tsb_v, sem2)
        pltpu.make_async_copy(new_hbm.at[srcc_v], rows_v, sem).wait()
        pltpu.make_async_copy(ts_hbm.at[srcc_v], tsb_v, sem2).wait()
        pltpu.async_copy(rows_v, outm_hbm.at[idxc_v], sem)
        pltpu.async_copy(tsb_v, outl_hbm.at[idxc_v], sem2)
        pltpu.make_async_copy(rows_v, outm_hbm.at[idxc_v], sem).wait()
        pltpu.make_async_copy(tsb_v, outl_hbm.at[idxc_v], sem2).wait()
        return carry
    lax.fori_loop(0, nch, _chunk, 0)


def kernel(memory, last_update, unique_node_ids, unique_messages, timestamps,
           W_ih, W_hh, b_ih, b_hh, ln_gamma, ln_beta):
    ids = unique_node_ids.astype(jnp.int32)
    cur = _sc_gather(memory, ids)
    new_mem = _tc_gru(cur, unique_messages, W_ih, W_hh,
                      b_ih[None, :], b_hh[None, :],
                      ln_gamma[None, :], ln_beta[None, :])
    out_mem, out_lu = _sc_scatter(memory, last_update, ids, timestamps,
                                  new_mem)
    return out_mem, out_lu


# speedup vs baseline: 1.0073x; 1.0073x over previous
import functools

import jax
import jax.numpy as jnp
from jax import lax
from jax.experimental import pallas as pl
from jax.experimental.pallas import tpu as pltpu
from jax.experimental.pallas import tpu_sc as plsc

M = 100000
D = 128
DMSG = 256
B = 16384
NW = 32
BPW = B // NW
RNG = 3136
RNG_LAST = M - (NW - 1) * RNG
NVREG = B // 16
NTAB = RNG // 16
CAP = RNG + 64

_MESH = dict(core_axis_name="c", subcore_axis_name="s", num_cores=2,
             num_subcores=16)


def _wid():
    return lax.axis_index("s") * 2 + lax.axis_index("c")


def _lane_iota():
    return lax.iota(jnp.int32, 16)


def _shift_up(x):
    idx = jnp.minimum(_lane_iota() + 1, 15)
    dn = lax.GatherDimensionNumbers(
        offset_dims=(), collapsed_slice_dims=(0,), start_index_map=(0,))
    return lax.gather(x, idx[:, None], dn, (1,),
                      mode=lax.GatherScatterMode.PROMISE_IN_BOUNDS)


def _splat0(x):
    idx = jnp.zeros((16,), jnp.int32)
    dn = lax.GatherDimensionNumbers(
        offset_dims=(), collapsed_slice_dims=(0,), start_index_map=(0,))
    return lax.gather(x, idx[:, None], dn, (1,),
                      mode=lax.GatherScatterMode.PROMISE_IN_BOUNDS)


@functools.partial(
    pl.kernel,
    out_type=jax.ShapeDtypeStruct((B, D), jnp.float32),
    mesh=plsc.VectorSubcoreMesh(**_MESH),
    scratch_types=[
        pltpu.VMEM((BPW,), jnp.int32),
        pltpu.VMEM((BPW, D), jnp.float32),
        pltpu.SemaphoreType.DMA,
    ],
)
def _sc_gather(mem_hbm, ids_hbm, cur_hbm, idx_v, rows_v, sem):
    base = _wid() * BPW
    pltpu.sync_copy(ids_hbm.at[pl.ds(base, BPW)], idx_v)
    for j in range(BPW // 128):
        pltpu.async_copy(mem_hbm.at[idx_v.at[pl.ds(j * 128, 128)]],
                         rows_v.at[pl.ds(j * 128, 128)], sem)
    for j in range(BPW // 128):
        pltpu.make_async_copy(mem_hbm.at[idx_v.at[pl.ds(j * 128, 128)]],
                              rows_v.at[pl.ds(j * 128, 128)], sem).wait()
    pltpu.sync_copy(rows_v, cur_hbm.at[pl.ds(base, BPW)])


def _gru_body(msg_ref, cur_ref, wih_ref, whh_ref, bih_ref, bhh_ref,
              g_ref, bt_ref, out_ref):
    msg = msg_ref[...]
    cur = cur_ref[...]
    dn = (((1,), (1,)), ((), ()))
    gi = lax.dot_general(msg, wih_ref[...], dn,
                         preferred_element_type=jnp.float32) + bih_ref[...]
    gh = lax.dot_general(cur, whh_ref[...], dn,
                         preferred_element_type=jnp.float32) + bhh_ref[...]
    r = jax.nn.sigmoid(gi[:, :D] + gh[:, :D])
    z = jax.nn.sigmoid(gi[:, D:2 * D] + gh[:, D:2 * D])
    n = jnp.tanh(gi[:, 2 * D:] + r * gh[:, 2 * D:])
    h = (1.0 - z) * n + z * cur
    mu = jnp.mean(h, axis=-1, keepdims=True)
    var = jnp.mean((h - mu) ** 2, axis=-1, keepdims=True)
    out_ref[...] = (h - mu) * lax.rsqrt(var + 1e-5) * g_ref[...] + bt_ref[...]


_GRU_BLK = 512


def _tc_gru(cur, msgs, W_ih, W_hh, b_ih, b_hh, g, bt):
    grid = B // _GRU_BLK
    return pl.pallas_call(
        _gru_body,
        grid=(grid,),
        in_specs=[
            pl.BlockSpec((_GRU_BLK, DMSG), lambda i: (i, 0)),
            pl.BlockSpec((_GRU_BLK, D), lambda i: (i, 0)),
            pl.BlockSpec((3 * D, DMSG), lambda i: (0, 0)),
            pl.BlockSpec((3 * D, D), lambda i: (0, 0)),
            pl.BlockSpec((1, 3 * D), lambda i: (0, 0)),
            pl.BlockSpec((1, 3 * D), lambda i: (0, 0)),
            pl.BlockSpec((1, D), lambda i: (0, 0)),
            pl.BlockSpec((1, D), lambda i: (0, 0)),
        ],
        out_specs=pl.BlockSpec((_GRU_BLK, D), lambda i: (i, 0)),
        out_shape=jax.ShapeDtypeStruct((B, D), jnp.float32),
    )(msgs, cur, W_ih, W_hh, b_ih, b_hh, g, bt)


_NCB = 3
_CPR = 128


def _emit_range_copy(src, dst, base, n_chunks, tail_rows, bufs, in_sems,
                     out_sems, compute_segment):
    def _in(k):
        off = base + k * _CPR
        return pltpu.make_async_copy(src.at[pl.ds(off, _CPR)],
                                     bufs.at[k % _NCB], in_sems.at[k % _NCB])

    def _out(k):
        off = base + k * _CPR
        return pltpu.make_async_copy(bufs.at[k % _NCB],
                                     dst.at[pl.ds(off, _CPR)],
                                     out_sems.at[k % _NCB])

    waited = set()
    for k in range(min(_NCB, n_chunks)):
        _in(k).start()
    for k in range(n_chunks):
        compute_segment(k, n_chunks)
        j = k - 2
        if j >= 0 and j + _NCB < n_chunks:
            _out(j).wait()
            waited.add(j)
            _in(j + _NCB).start()
        _in(k).wait()
        _out(k).start()
    for k in range(n_chunks):
        if k not in waited:
            _out(k).wait()
    if tail_rows:
        off = base + n_chunks * _CPR
        pltpu.sync_copy(src.at[pl.ds(off, tail_rows)],
                        bufs.at[0, pl.ds(0, tail_rows)])
        pltpu.sync_copy(bufs.at[0, pl.ds(0, tail_rows)],
                        dst.at[pl.ds(off, tail_rows)])


@functools.partial(
    pl.kernel,
    out_type=(jax.ShapeDtypeStruct((M, D), jnp.float32),
              jax.ShapeDtypeStruct((M,), jnp.float32)),
    mesh=plsc.VectorSubcoreMesh(**_MESH),
    scratch_types=[
        pltpu.VMEM((B,), jnp.int32),
        pltpu.VMEM((RNG,), jnp.int32),
        pltpu.VMEM((CAP,), jnp.int32),
        pltpu.VMEM((CAP,), jnp.int32),
        pltpu.VMEM((2, 128), jnp.int32),
        pltpu.VMEM((2, 128), jnp.int32),
        pltpu.VMEM((2, 128, D), jnp.float32),
        pltpu.VMEM((2, 128), jnp.float32),
        pltpu.VMEM((_NCB, _CPR, D), jnp.float32),
        pltpu.VMEM((RNG,), jnp.float32),
        pltpu.SemaphoreType.DMA((_NCB,)),
        pltpu.SemaphoreType.DMA((_NCB,)),
        pltpu.SemaphoreType.DMA((2,)),
        pltpu.SemaphoreType.DMA((2,)),
        pltpu.SemaphoreType.DMA((2,)),
        pltpu.SemaphoreType.DMA((2,)),
    ],
    compiler_params=pltpu.CompilerParams(needs_layout_passes=False),
)
def _sc_scatter(mem_hbm, lu_hbm, ids_hbm, ts_hbm, new_hbm,
                outm_hbm, outl_hbm,
                ids_v, tab_v, dstf_v, srcf_v, idxc_v, srcc_v, rows_v, tsb_v,
                cpb_v, lub_v, in_sems, out_sems, gr_sems, gt_sems, sr_sems,
                st_sems):
    wid = _wid()
    base = wid * RNG
    lane = _lane_iota()

    pltpu.sync_copy(ids_hbm, ids_v)

    def _clear(t, carry):
        tab_v[pl.ds(t * 16, 16)] = jnp.full((16,), -1, jnp.int32)
        return carry
    lax.fori_loop(0, NTAB, _clear, 0)

    _UNR = 4

    def _scan(g, carry):
        comps = []
        for u in range(_UNR):
            v = g * _UNR + u
            ids16 = ids_v[pl.ds(v * 16, 16)]
            comps.append(plsc.sort_key_val(
                ids16 * 16384 + (v * 16 + lane),
                ids16 * 16384 + (v * 16 + lane))[0])
        for comp in comps:
            sid = lax.shift_right_logical(comp, 14)
            si = lax.bitwise_and(comp, 16383)
            nxt = _shift_up(sid)
            m = ((sid >= base) & (sid < base + RNG)
                 & ((sid != nxt) | (lane == 15)))
            plsc.store_scatter(tab_v, [sid - base], si, mask=m)
        return carry

    def _scan_segment(k, n_chunks):
        ngrp = NVREG // _UNR
        lo = k * ngrp // n_chunks
        hi = (k + 1) * ngrp // n_chunks
        lax.fori_loop(lo, hi, _scan, 0)

    @pl.when(wid < NW - 1)
    def _():
        _emit_range_copy(mem_hbm, outm_hbm, base, RNG // _CPR, RNG % _CPR,
                         cpb_v, in_sems, out_sems, _scan_segment)
        pltpu.sync_copy(lu_hbm.at[pl.ds(base, RNG)], lub_v)
        pltpu.sync_copy(lub_v, outl_hbm.at[pl.ds(base, RNG)])

    @pl.when(wid == NW - 1)
    def _():
        _emit_range_copy(mem_hbm, outm_hbm, base, RNG_LAST // _CPR,
                         RNG_LAST % _CPR, cpb_v, in_sems, out_sems,
                         _scan_segment)
        pltpu.sync_copy(lu_hbm.at[pl.ds(base, RNG_LAST)],
                        lub_v.at[pl.ds(0, RNG_LAST)])
        pltpu.sync_copy(lub_v.at[pl.ds(0, RNG_LAST)],
                        outl_hbm.at[pl.ds(base, RNG_LAST)])

    def _compact(t, off):
        tv = tab_v[pl.ds(t * 16, 16)]
        m = tv >= 0
        pc = plsc.cumsum(jnp.where(m, 1, 0).astype(jnp.int32))
        tgt = off + pc - 1
        plsc.store_scatter(dstf_v, [tgt], base + t * 16 + lane, mask=m)
        plsc.store_scatter(srcf_v, [tgt], tv, mask=m)
        return off + jnp.max(plsc.all_reduce_population_count(m))
    cnt = lax.fori_loop(0, NTAB, _compact, jnp.int32(0))

    nch = (cnt + 127) // 128

    @pl.when(cnt > 0)
    def _():
        pad_end = nch * 128

        def _pad(k, carry):
            pos = cnt + k * 16 + lane
            pm = pos < pad_end
            j = jnp.minimum(pos - cnt, cnt - 1)
            dv = plsc.load_gather(dstf_v, [j])
            sv = plsc.load_gather(srcf_v, [j])
            plsc.store_scatter(dstf_v, [pos], dv, mask=pm)
            plsc.store_scatter(srcf_v, [pos], sv, mask=pm)
            return carry
        lax.fori_loop(0, 8, _pad, 0)

    def _g_rows(s):
        return pltpu.make_async_copy(new_hbm.at[srcc_v.at[s]],
                                     rows_v.at[s], gr_sems.at[s])

    def _g_ts(s):
        return pltpu.make_async_copy(ts_hbm.at[srcc_v.at[s]],
                                     tsb_v.at[s], gt_sems.at[s])

    def _s_rows(s):
        return pltpu.make_async_copy(rows_v.at[s],
                                     outm_hbm.at[idxc_v.at[s]], sr_sems.at[s])

    def _s_ts(s):
        return pltpu.make_async_copy(tsb_v.at[s],
                                     outl_hbm.at[idxc_v.at[s]], st_sems.at[s])

    def _pipe(c, carry):
        slot = lax.bitwise_and(c, 1)

        @pl.when(c < nch)
        def _():
            @pl.when(c >= 2)
            def _():
                _s_rows(slot).wait()
                _s_ts(slot).wait()

            def _fill(j, carry2):
                idxc_v[slot, pl.ds(j * 16, 16)] = (
                    dstf_v[pl.ds(c * 128 + j * 16, 16)])
                srcc_v[slot, pl.ds(j * 16, 16)] = (
                    srcf_v[pl.ds(c * 128 + j * 16, 16)])
                return carry2
            lax.fori_loop(0, 8, _fill, 0)
            _g_rows(slot).start()
            _g_ts(slot).start()

        @pl.when(c >= 1)
        def _():
            pslot = lax.bitwise_and(c - 1, 1)
            _g_rows(pslot).wait()
            _g_ts(pslot).wait()
            _s_rows(pslot).start()
            _s_ts(pslot).start()
        return carry
    lax.fori_loop(0, nch + 1, _pipe, 0)

    @pl.when(nch >= 2)
    def _():
        s = lax.bitwise_and(nch, 1)
        _s_rows(s).wait()
        _s_ts(s).wait()

    @pl.when(nch >= 1)
    def _():
        s = lax.bitwise_and(nch - 1, 1)
        _s_rows(s).wait()
        _s_ts(s).wait()


def kernel(memory, last_update, unique_node_ids, unique_messages, timestamps,
           W_ih, W_hh, b_ih, b_hh, ln_gamma, ln_beta):
    ids = unique_node_ids.astype(jnp.int32)
    cur = _sc_gather(memory, ids)
    new_mem = _tc_gru(cur, unique_messages, W_ih, W_hh,
                      b_ih[None, :], b_hh[None, :],
                      ln_gamma[None, :], ln_beta[None, :])
    out_mem, out_lu = _sc_scatter(memory, last_update, ids, timestamps,
                                  new_mem)
    return out_mem, out_lu


# speedup vs baseline: 1.3430x; 1.3333x over previous
import functools

import jax
import jax.numpy as jnp
from jax import lax
from jax.experimental import pallas as pl
from jax.experimental.pallas import tpu as pltpu
from jax.experimental.pallas import tpu_sc as plsc

M = 100000
D = 128
DMSG = 256
B = 16384
NW = 32
BPW = B // NW
RNG = 3136
RNG_LAST = M - (NW - 1) * RNG
NVREG = B // 16
NTAB = RNG // 16
CAP = RNG + 64

_MESH = dict(core_axis_name="c", subcore_axis_name="s", num_cores=2,
             num_subcores=16)


def _wid():
    return lax.axis_index("s") * 2 + lax.axis_index("c")


def _lane_iota():
    return lax.iota(jnp.int32, 16)


def _shift_up(x):
    idx = jnp.minimum(_lane_iota() + 1, 15)
    dn = lax.GatherDimensionNumbers(
        offset_dims=(), collapsed_slice_dims=(0,), start_index_map=(0,))
    return lax.gather(x, idx[:, None], dn, (1,),
                      mode=lax.GatherScatterMode.PROMISE_IN_BOUNDS)


def _splat0(x):
    idx = jnp.zeros((16,), jnp.int32)
    dn = lax.GatherDimensionNumbers(
        offset_dims=(), collapsed_slice_dims=(0,), start_index_map=(0,))
    return lax.gather(x, idx[:, None], dn, (1,),
                      mode=lax.GatherScatterMode.PROMISE_IN_BOUNDS)


@functools.partial(
    pl.kernel,
    out_type=jax.ShapeDtypeStruct((B, D), jnp.float32),
    mesh=plsc.VectorSubcoreMesh(**_MESH),
    scratch_types=[
        pltpu.VMEM((BPW,), jnp.int32),
        pltpu.VMEM((BPW, D), jnp.float32),
        pltpu.SemaphoreType.DMA,
    ],
)
def _sc_gather(mem_hbm, ids_hbm, cur_hbm, idx_v, rows_v, sem):
    base = _wid() * BPW
    pltpu.sync_copy(ids_hbm.at[pl.ds(base, BPW)], idx_v)
    for j in range(BPW // 128):
        pltpu.async_copy(mem_hbm.at[idx_v.at[pl.ds(j * 128, 128)]],
                         rows_v.at[pl.ds(j * 128, 128)], sem)
    for j in range(BPW // 128):
        pltpu.make_async_copy(mem_hbm.at[idx_v.at[pl.ds(j * 128, 128)]],
                              rows_v.at[pl.ds(j * 128, 128)], sem).wait()
    pltpu.sync_copy(rows_v, cur_hbm.at[pl.ds(base, BPW)])


def _gru_body(msg_ref, cur_ref, wih_ref, whh_ref, bih_ref, bhh_ref,
              g_ref, bt_ref, out_ref):
    msg = msg_ref[...]
    cur = cur_ref[...]
    dn = (((1,), (1,)), ((), ()))
    gi = lax.dot_general(msg, wih_ref[...], dn,
                         preferred_element_type=jnp.float32) + bih_ref[...]
    gh = lax.dot_general(cur, whh_ref[...], dn,
                         preferred_element_type=jnp.float32) + bhh_ref[...]
    r = jax.nn.sigmoid(gi[:, :D] + gh[:, :D])
    z = jax.nn.sigmoid(gi[:, D:2 * D] + gh[:, D:2 * D])
    n = jnp.tanh(gi[:, 2 * D:] + r * gh[:, 2 * D:])
    h = (1.0 - z) * n + z * cur
    mu = jnp.mean(h, axis=-1, keepdims=True)
    var = jnp.mean((h - mu) ** 2, axis=-1, keepdims=True)
    out_ref[...] = (h - mu) * lax.rsqrt(var + 1e-5) * g_ref[...] + bt_ref[...]


_GRU_BLK = 512


def _tc_gru(cur, msgs, W_ih, W_hh, b_ih, b_hh, g, bt):
    grid = B // _GRU_BLK
    return pl.pallas_call(
        _gru_body,
        grid=(grid,),
        in_specs=[
            pl.BlockSpec((_GRU_BLK, DMSG), lambda i: (i, 0)),
            pl.BlockSpec((_GRU_BLK, D), lambda i: (i, 0)),
            pl.BlockSpec((3 * D, DMSG), lambda i: (0, 0)),
            pl.BlockSpec((3 * D, D), lambda i: (0, 0)),
            pl.BlockSpec((1, 3 * D), lambda i: (0, 0)),
            pl.BlockSpec((1, 3 * D), lambda i: (0, 0)),
            pl.BlockSpec((1, D), lambda i: (0, 0)),
            pl.BlockSpec((1, D), lambda i: (0, 0)),
        ],
        out_specs=pl.BlockSpec((_GRU_BLK, D), lambda i: (i, 0)),
        out_shape=jax.ShapeDtypeStruct((B, D), jnp.float32),
    )(msgs, cur, W_ih, W_hh, b_ih, b_hh, g, bt)


_NCB = 3
_CPR = 128


def _emit_range_copy(src, dst, base, n_chunks, tail_rows, bufs, in_sems,
                     out_sems, compute_segment):
    def _in(k):
        off = base + k * _CPR
        return pltpu.make_async_copy(src.at[pl.ds(off, _CPR)],
                                     bufs.at[k % _NCB], in_sems.at[k % _NCB])

    def _out(k):
        off = base + k * _CPR
        return pltpu.make_async_copy(bufs.at[k % _NCB],
                                     dst.at[pl.ds(off, _CPR)],
                                     out_sems.at[k % _NCB])

    waited = set()
    for k in range(min(_NCB, n_chunks)):
        _in(k).start()
    for k in range(n_chunks):
        compute_segment(k, n_chunks)
        j = k - 2
        if j >= 0 and j + _NCB < n_chunks:
            _out(j).wait()
            waited.add(j)
            _in(j + _NCB).start()
        _in(k).wait()
        _out(k).start()
    for k in range(n_chunks):
        if k not in waited:
            _out(k).wait()
    if tail_rows:
        off = base + n_chunks * _CPR
        pltpu.sync_copy(src.at[pl.ds(off, tail_rows)],
                        bufs.at[0, pl.ds(0, tail_rows)])
        pltpu.sync_copy(bufs.at[0, pl.ds(0, tail_rows)],
                        dst.at[pl.ds(off, tail_rows)])


@functools.partial(
    pl.kernel,
    out_type=(jax.ShapeDtypeStruct((M, D), jnp.float32),
              jax.ShapeDtypeStruct((M,), jnp.float32)),
    mesh=plsc.VectorSubcoreMesh(**_MESH),
    scratch_types=[
        pltpu.VMEM((B,), jnp.int32),
        pltpu.VMEM((RNG,), jnp.int32),
        pltpu.VMEM((CAP,), jnp.int32),
        pltpu.VMEM((CAP,), jnp.int32),
        pltpu.VMEM((2, 128), jnp.int32),
        pltpu.VMEM((2, 128), jnp.int32),
        pltpu.VMEM((2, 128, D), jnp.float32),
        pltpu.VMEM((B,), jnp.float32),
        pltpu.VMEM((_NCB, _CPR, D), jnp.float32),
        pltpu.VMEM((RNG,), jnp.float32),
        pltpu.SemaphoreType.DMA((_NCB,)),
        pltpu.SemaphoreType.DMA((_NCB,)),
        pltpu.SemaphoreType.DMA((2,)),
        pltpu.SemaphoreType.DMA((2,)),
    ],
    compiler_params=pltpu.CompilerParams(needs_layout_passes=False),
)
def _sc_scatter(mem_hbm, lu_hbm, ids_hbm, ts_hbm, new_hbm,
                outm_hbm, outl_hbm,
                ids_v, tab_v, dstf_v, srcf_v, idxc_v, srcc_v, rows_v, tsall_v,
                cpb_v, lub_v, in_sems, out_sems, gr_sems, sr_sems):
    wid = _wid()
    base = wid * RNG
    lane = _lane_iota()

    pltpu.sync_copy(ids_hbm, ids_v)
    pltpu.sync_copy(ts_hbm, tsall_v)

    def _clear(t, carry):
        tab_v[pl.ds(t * 16, 16)] = jnp.full((16,), -1, jnp.int32)
        return carry
    lax.fori_loop(0, NTAB, _clear, 0)

    _UNR = 4

    def _scan(g, carry):
        comps = []
        for u in range(_UNR):
            v = g * _UNR + u
            ids16 = ids_v[pl.ds(v * 16, 16)]
            comps.append(plsc.sort_key_val(
                ids16 * 16384 + (v * 16 + lane),
                ids16 * 16384 + (v * 16 + lane))[0])
        for comp in comps:
            sid = lax.shift_right_logical(comp, 14)
            si = lax.bitwise_and(comp, 16383)
            nxt = _shift_up(sid)
            m = ((sid >= base) & (sid < base + RNG)
                 & ((sid != nxt) | (lane == 15)))
            plsc.store_scatter(tab_v, [sid - base], si, mask=m)
        return carry

    def _scan_segment(k, n_chunks):
        ngrp = NVREG // _UNR
        lo = k * ngrp // n_chunks
        hi = (k + 1) * ngrp // n_chunks
        lax.fori_loop(lo, hi, _scan, 0)

    @pl.when(wid < NW - 1)
    def _():
        _emit_range_copy(mem_hbm, outm_hbm, base, RNG // _CPR, RNG % _CPR,
                         cpb_v, in_sems, out_sems, _scan_segment)
        pltpu.sync_copy(lu_hbm.at[pl.ds(base, RNG)], lub_v)

    @pl.when(wid == NW - 1)
    def _():
        _emit_range_copy(mem_hbm, outm_hbm, base, RNG_LAST // _CPR,
                         RNG_LAST % _CPR, cpb_v, in_sems, out_sems,
                         _scan_segment)
        pltpu.sync_copy(lu_hbm.at[pl.ds(base, RNG_LAST)],
                        lub_v.at[pl.ds(0, RNG_LAST)])

    def _compact(t, off):
        tv = tab_v[pl.ds(t * 16, 16)]
        m = tv >= 0
        pc = plsc.cumsum(jnp.where(m, 1, 0).astype(jnp.int32))
        tgt = off + pc - 1
        plsc.store_scatter(dstf_v, [tgt], base + t * 16 + lane, mask=m)
        plsc.store_scatter(srcf_v, [tgt], tv, mask=m)
        return off + jnp.max(plsc.all_reduce_population_count(m))
    cnt = lax.fori_loop(0, NTAB, _compact, jnp.int32(0))

    nch = (cnt + 127) // 128

    @pl.when(cnt > 0)
    def _():
        pad_end = nch * 128

        def _pad(k, carry):
            pos = cnt + k * 16 + lane
            pm = pos < pad_end
            j = jnp.minimum(pos - cnt, cnt - 1)
            dv = plsc.load_gather(dstf_v, [j])
            sv = plsc.load_gather(srcf_v, [j])
            plsc.store_scatter(dstf_v, [pos], dv, mask=pm)
            plsc.store_scatter(srcf_v, [pos], sv, mask=pm)
            return carry
        lax.fori_loop(0, 8, _pad, 0)

    def _g_rows(s):
        return pltpu.make_async_copy(new_hbm.at[srcc_v.at[s]],
                                     rows_v.at[s], gr_sems.at[s])

    def _s_rows(s):
        return pltpu.make_async_copy(rows_v.at[s],
                                     outm_hbm.at[idxc_v.at[s]], sr_sems.at[s])

    def _pipe(c, carry):
        slot = lax.bitwise_and(c, 1)

        @pl.when(c < nch)
        def _():
            @pl.when(c >= 2)
            def _():
                _s_rows(slot).wait()

            def _fill(j, carry2):
                idxc_v[slot, pl.ds(j * 16, 16)] = (
                    dstf_v[pl.ds(c * 128 + j * 16, 16)])
                srcc_v[slot, pl.ds(j * 16, 16)] = (
                    srcf_v[pl.ds(c * 128 + j * 16, 16)])
                return carry2
            lax.fori_loop(0, 8, _fill, 0)
            _g_rows(slot).start()

        @pl.when(c >= 1)
        def _():
            pslot = lax.bitwise_and(c - 1, 1)
            _g_rows(pslot).wait()
            _s_rows(pslot).start()
        return carry
    lax.fori_loop(0, nch + 1, _pipe, 0)

    def _tsap(q, carry):
        pos = q * 16 + lane
        d16 = plsc.load_gather(dstf_v, [pos])
        s16 = plsc.load_gather(srcf_v, [pos])
        tsv = plsc.load_gather(tsall_v, [s16])
        plsc.store_scatter(lub_v, [d16 - base], tsv)
        return carry
    lax.fori_loop(0, nch * 8, _tsap, 0)

    @pl.when(wid < NW - 1)
    def _():
        pltpu.sync_copy(lub_v, outl_hbm.at[pl.ds(base, RNG)])

    @pl.when(wid == NW - 1)
    def _():
        pltpu.sync_copy(lub_v.at[pl.ds(0, RNG_LAST)],
                        outl_hbm.at[pl.ds(base, RNG_LAST)])

    @pl.when(nch >= 2)
    def _():
        _s_rows(lax.bitwise_and(nch, 1)).wait()

    @pl.when(nch >= 1)
    def _():
        _s_rows(lax.bitwise_and(nch - 1, 1)).wait()


def kernel(memory, last_update, unique_node_ids, unique_messages, timestamps,
           W_ih, W_hh, b_ih, b_hh, ln_gamma, ln_beta):
    ids = unique_node_ids.astype(jnp.int32)
    cur = _sc_gather(memory, ids)
    new_mem = _tc_gru(cur, unique_messages, W_ih, W_hh,
                      b_ih[None, :], b_hh[None, :],
                      ln_gamma[None, :], ln_beta[None, :])
    out_mem, out_lu = _sc_scatter(memory, last_update, ids, timestamps,
                                  new_mem)
    return out_mem, out_lu


# speedup vs baseline: 1.4610x; 1.0878x over previous
import functools

import jax
import jax.numpy as jnp
from jax import lax
from jax.experimental import pallas as pl
from jax.experimental.pallas import tpu as pltpu
from jax.experimental.pallas import tpu_sc as plsc

M = 100000
D = 128
DMSG = 256
B = 16384
NW = 32
BPW = B // NW
RNG = 3136
RNG_LAST = M - (NW - 1) * RNG
NVREG = B // 16
NTAB = RNG // 16
CAP = RNG + 64

_MESH = dict(core_axis_name="c", subcore_axis_name="s", num_cores=2,
             num_subcores=16)


def _wid():
    return lax.axis_index("s") * 2 + lax.axis_index("c")


def _lane_iota():
    return lax.iota(jnp.int32, 16)


def _shift_up(x):
    idx = jnp.minimum(_lane_iota() + 1, 15)
    dn = lax.GatherDimensionNumbers(
        offset_dims=(), collapsed_slice_dims=(0,), start_index_map=(0,))
    return lax.gather(x, idx[:, None], dn, (1,),
                      mode=lax.GatherScatterMode.PROMISE_IN_BOUNDS)


def _splat0(x):
    idx = jnp.zeros((16,), jnp.int32)
    dn = lax.GatherDimensionNumbers(
        offset_dims=(), collapsed_slice_dims=(0,), start_index_map=(0,))
    return lax.gather(x, idx[:, None], dn, (1,),
                      mode=lax.GatherScatterMode.PROMISE_IN_BOUNDS)


@functools.partial(
    pl.kernel,
    out_type=jax.ShapeDtypeStruct((B, D), jnp.float32),
    mesh=plsc.VectorSubcoreMesh(**_MESH),
    scratch_types=[
        pltpu.VMEM((BPW,), jnp.int32),
        pltpu.VMEM((BPW, D), jnp.float32),
        pltpu.SemaphoreType.DMA,
    ],
)
def _sc_gather(mem_hbm, ids_hbm, cur_hbm, idx_v, rows_v, sem):
    base = _wid() * BPW
    pltpu.sync_copy(ids_hbm.at[pl.ds(base, BPW)], idx_v)
    for j in range(BPW // 128):
        pltpu.async_copy(mem_hbm.at[idx_v.at[pl.ds(j * 128, 128)]],
                         rows_v.at[pl.ds(j * 128, 128)], sem)
    for j in range(BPW // 128):
        pltpu.make_async_copy(mem_hbm.at[idx_v.at[pl.ds(j * 128, 128)]],
                              rows_v.at[pl.ds(j * 128, 128)], sem).wait()
    pltpu.sync_copy(rows_v, cur_hbm.at[pl.ds(base, BPW)])


def _gru_body(msg_ref, cur_ref, wih_ref, whh_ref, bih_ref, bhh_ref,
              g_ref, bt_ref, out_ref):
    msg = msg_ref[...]
    cur = cur_ref[...]
    dn = (((1,), (1,)), ((), ()))
    gi = lax.dot_general(msg, wih_ref[...], dn,
                         preferred_element_type=jnp.float32) + bih_ref[...]
    gh = lax.dot_general(cur, whh_ref[...], dn,
                         preferred_element_type=jnp.float32) + bhh_ref[...]
    r = jax.nn.sigmoid(gi[:, :D] + gh[:, :D])
    z = jax.nn.sigmoid(gi[:, D:2 * D] + gh[:, D:2 * D])
    n = jnp.tanh(gi[:, 2 * D:] + r * gh[:, 2 * D:])
    h = (1.0 - z) * n + z * cur
    mu = jnp.mean(h, axis=-1, keepdims=True)
    var = jnp.mean((h - mu) ** 2, axis=-1, keepdims=True)
    out_ref[...] = (h - mu) * lax.rsqrt(var + 1e-5) * g_ref[...] + bt_ref[...]


_GRU_BLK = 1024


def _tc_gru(cur, msgs, W_ih, W_hh, b_ih, b_hh, g, bt):
    grid = B // _GRU_BLK
    return pl.pallas_call(
        _gru_body,
        grid=(grid,),
        in_specs=[
            pl.BlockSpec((_GRU_BLK, DMSG), lambda i: (i, 0)),
            pl.BlockSpec((_GRU_BLK, D), lambda i: (i, 0)),
            pl.BlockSpec((3 * D, DMSG), lambda i: (0, 0)),
            pl.BlockSpec((3 * D, D), lambda i: (0, 0)),
            pl.BlockSpec((1, 3 * D), lambda i: (0, 0)),
            pl.BlockSpec((1, 3 * D), lambda i: (0, 0)),
            pl.BlockSpec((1, D), lambda i: (0, 0)),
            pl.BlockSpec((1, D), lambda i: (0, 0)),
        ],
        out_specs=pl.BlockSpec((_GRU_BLK, D), lambda i: (i, 0)),
        out_shape=jax.ShapeDtypeStruct((B, D), jnp.float32),
    )(msgs, cur, W_ih, W_hh, b_ih, b_hh, g, bt)


_NCB = 3
_CPR = 128


def _emit_range_copy(src, dst, base, n_chunks, tail_rows, bufs, in_sems,
                     out_sems, compute_segment):
    def _in(k):
        off = base + k * _CPR
        return pltpu.make_async_copy(src.at[pl.ds(off, _CPR)],
                                     bufs.at[k % _NCB], in_sems.at[k % _NCB])

    def _out(k):
        off = base + k * _CPR
        return pltpu.make_async_copy(bufs.at[k % _NCB],
                                     dst.at[pl.ds(off, _CPR)],
                                     out_sems.at[k % _NCB])

    waited = set()
    for k in range(min(_NCB, n_chunks)):
        _in(k).start()
    for k in range(n_chunks):
        compute_segment(k, n_chunks)
        j = k - 2
        if j >= 0 and j + _NCB < n_chunks:
            _out(j).wait()
            waited.add(j)
            _in(j + _NCB).start()
        _in(k).wait()
        _out(k).start()
    for k in range(n_chunks):
        if k not in waited:
            _out(k).wait()
    if tail_rows:
        off = base + n_chunks * _CPR
        pltpu.sync_copy(src.at[pl.ds(off, tail_rows)],
                        bufs.at[0, pl.ds(0, tail_rows)])
        pltpu.sync_copy(bufs.at[0, pl.ds(0, tail_rows)],
                        dst.at[pl.ds(off, tail_rows)])


@functools.partial(
    pl.kernel,
    out_type=(jax.ShapeDtypeStruct((M, D), jnp.float32),
              jax.ShapeDtypeStruct((M,), jnp.float32)),
    mesh=plsc.VectorSubcoreMesh(**_MESH),
    scratch_types=[
        pltpu.VMEM((B,), jnp.int32),
        pltpu.VMEM((RNG,), jnp.int32),
        pltpu.VMEM((CAP,), jnp.int32),
        pltpu.VMEM((CAP,), jnp.int32),
        pltpu.VMEM((2, 128), jnp.int32),
        pltpu.VMEM((2, 128), jnp.int32),
        pltpu.VMEM((2, 128, D), jnp.float32),
        pltpu.VMEM((B,), jnp.float32),
        pltpu.VMEM((_NCB, _CPR, D), jnp.float32),
        pltpu.VMEM((RNG,), jnp.float32),
        pltpu.SemaphoreType.DMA((_NCB,)),
        pltpu.SemaphoreType.DMA((_NCB,)),
        pltpu.SemaphoreType.DMA((2,)),
        pltpu.SemaphoreType.DMA((2,)),
    ],
    compiler_params=pltpu.CompilerParams(needs_layout_passes=False),
)
def _sc_scatter(mem_hbm, lu_hbm, ids_hbm, ts_hbm, new_hbm,
                outm_hbm, outl_hbm,
                ids_v, tab_v, dstf_v, srcf_v, idxc_v, srcc_v, rows_v, tsall_v,
                cpb_v, lub_v, in_sems, out_sems, gr_sems, sr_sems):
    wid = _wid()
    base = wid * RNG
    lane = _lane_iota()

    pltpu.sync_copy(ids_hbm, ids_v)
    pltpu.sync_copy(ts_hbm, tsall_v)

    def _clear(t, carry):
        tab_v[pl.ds(t * 16, 16)] = jnp.full((16,), -1, jnp.int32)
        return carry
    lax.fori_loop(0, NTAB, _clear, 0)

    _UNR = 4

    def _scan(g, carry):
        comps = []
        for u in range(_UNR):
            v = g * _UNR + u
            ids16 = ids_v[pl.ds(v * 16, 16)]
            comps.append(plsc.sort_key_val(
                ids16 * 16384 + (v * 16 + lane),
                ids16 * 16384 + (v * 16 + lane))[0])
        for comp in comps:
            sid = lax.shift_right_logical(comp, 14)
            si = lax.bitwise_and(comp, 16383)
            nxt = _shift_up(sid)
            m = ((sid >= base) & (sid < base + RNG)
                 & ((sid != nxt) | (lane == 15)))
            plsc.store_scatter(tab_v, [sid - base], si, mask=m)
        return carry

    def _scan_segment(k, n_chunks):
        ngrp = NVREG // _UNR
        lo = k * ngrp // n_chunks
        hi = (k + 1) * ngrp // n_chunks
        lax.fori_loop(lo, hi, _scan, 0)

    @pl.when(wid < NW - 1)
    def _():
        _emit_range_copy(mem_hbm, outm_hbm, base, RNG // _CPR, RNG % _CPR,
                         cpb_v, in_sems, out_sems, _scan_segment)
        pltpu.sync_copy(lu_hbm.at[pl.ds(base, RNG)], lub_v)

    @pl.when(wid == NW - 1)
    def _():
        _emit_range_copy(mem_hbm, outm_hbm, base, RNG_LAST // _CPR,
                         RNG_LAST % _CPR, cpb_v, in_sems, out_sems,
                         _scan_segment)
        pltpu.sync_copy(lu_hbm.at[pl.ds(base, RNG_LAST)],
                        lub_v.at[pl.ds(0, RNG_LAST)])

    def _compact(t, off):
        tv = tab_v[pl.ds(t * 16, 16)]
        m = tv >= 0
        pc = plsc.cumsum(jnp.where(m, 1, 0).astype(jnp.int32))
        tgt = off + pc - 1
        plsc.store_scatter(dstf_v, [tgt], base + t * 16 + lane, mask=m)
        plsc.store_scatter(srcf_v, [tgt], tv, mask=m)
        return off + jnp.max(plsc.all_reduce_population_count(m))
    cnt = lax.fori_loop(0, NTAB, _compact, jnp.int32(0))

    nch = (cnt + 127) // 128

    @pl.when(cnt > 0)
    def _():
        pad_end = nch * 128

        def _pad(k, carry):
            pos = cnt + k * 16 + lane
            pm = pos < pad_end
            j = jnp.minimum(pos - cnt, cnt - 1)
            dv = plsc.load_gather(dstf_v, [j])
            sv = plsc.load_gather(srcf_v, [j])
            plsc.store_scatter(dstf_v, [pos], dv, mask=pm)
            plsc.store_scatter(srcf_v, [pos], sv, mask=pm)
            return carry
        lax.fori_loop(0, 8, _pad, 0)

    def _g_rows(s):
        return pltpu.make_async_copy(new_hbm.at[srcc_v.at[s]],
                                     rows_v.at[s], gr_sems.at[s])

    def _s_rows(s):
        return pltpu.make_async_copy(rows_v.at[s],
                                     outm_hbm.at[idxc_v.at[s]], sr_sems.at[s])

    def _pipe(c, carry):
        slot = lax.bitwise_and(c, 1)

        @pl.when(c < nch)
        def _():
            @pl.when(c >= 2)
            def _():
                _s_rows(slot).wait()

            def _fill(j, carry2):
                idxc_v[slot, pl.ds(j * 16, 16)] = (
                    dstf_v[pl.ds(c * 128 + j * 16, 16)])
                srcc_v[slot, pl.ds(j * 16, 16)] = (
                    srcf_v[pl.ds(c * 128 + j * 16, 16)])
                return carry2
            lax.fori_loop(0, 8, _fill, 0)
            _g_rows(slot).start()

        @pl.when(c >= 1)
        def _():
            pslot = lax.bitwise_and(c - 1, 1)
            _g_rows(pslot).wait()
            _s_rows(pslot).start()
        return carry
    lax.fori_loop(0, nch + 1, _pipe, 0)

    def _tsap(q, carry):
        pos = q * 16 + lane
        d16 = plsc.load_gather(dstf_v, [pos])
        s16 = plsc.load_gather(srcf_v, [pos])
        tsv = plsc.load_gather(tsall_v, [s16])
        plsc.store_scatter(lub_v, [d16 - base], tsv)
        return carry
    lax.fori_loop(0, nch * 8, _tsap, 0)

    @pl.when(wid < NW - 1)
    def _():
        pltpu.sync_copy(lub_v, outl_hbm.at[pl.ds(base, RNG)])

    @pl.when(wid == NW - 1)
    def _():
        pltpu.sync_copy(lub_v.at[pl.ds(0, RNG_LAST)],
                        outl_hbm.at[pl.ds(base, RNG_LAST)])

    @pl.when(nch >= 2)
    def _():
        _s_rows(lax.bitwise_and(nch, 1)).wait()

    @pl.when(nch >= 1)
    def _():
        _s_rows(lax.bitwise_and(nch - 1, 1)).wait()


def kernel(memory, last_update, unique_node_ids, unique_messages, timestamps,
           W_ih, W_hh, b_ih, b_hh, ln_gamma, ln_beta):
    ids = unique_node_ids.astype(jnp.int32)
    cur = _sc_gather(memory, ids)
    new_mem = _tc_gru(cur, unique_messages, W_ih, W_hh,
                      b_ih[None, :], b_hh[None, :],
                      ln_gamma[None, :], ln_beta[None, :])
    out_mem, out_lu = _sc_scatter(memory, last_update, ids, timestamps,
                                  new_mem)
    return out_mem, out_lu


# speedup vs baseline: 1.5139x; 1.0362x over previous
import functools

import jax
import jax.numpy as jnp
from jax import lax
from jax.experimental import pallas as pl
from jax.experimental.pallas import tpu as pltpu
from jax.experimental.pallas import tpu_sc as plsc

M = 100000
D = 128
DMSG = 256
B = 16384
NW = 32
BPW = B // NW
RNG = 3136
RNG_LAST = M - (NW - 1) * RNG
NVREG = B // 16
NTAB = RNG // 16
CAP = RNG + 64

_MESH = dict(core_axis_name="c", subcore_axis_name="s", num_cores=2,
             num_subcores=16)


def _wid():
    return lax.axis_index("s") * 2 + lax.axis_index("c")


def _lane_iota():
    return lax.iota(jnp.int32, 16)


def _shift_up(x):
    idx = jnp.minimum(_lane_iota() + 1, 15)
    dn = lax.GatherDimensionNumbers(
        offset_dims=(), collapsed_slice_dims=(0,), start_index_map=(0,))
    return lax.gather(x, idx[:, None], dn, (1,),
                      mode=lax.GatherScatterMode.PROMISE_IN_BOUNDS)


def _splat0(x):
    idx = jnp.zeros((16,), jnp.int32)
    dn = lax.GatherDimensionNumbers(
        offset_dims=(), collapsed_slice_dims=(0,), start_index_map=(0,))
    return lax.gather(x, idx[:, None], dn, (1,),
                      mode=lax.GatherScatterMode.PROMISE_IN_BOUNDS)


@functools.partial(
    pl.kernel,
    out_type=jax.ShapeDtypeStruct((B, D), jnp.float32),
    mesh=plsc.VectorSubcoreMesh(**_MESH),
    scratch_types=[
        pltpu.VMEM((BPW,), jnp.int32),
        pltpu.VMEM((BPW, D), jnp.float32),
        pltpu.SemaphoreType.DMA,
    ],
)
def _sc_gather(mem_hbm, ids_hbm, cur_hbm, idx_v, rows_v, sem):
    base = _wid() * BPW
    pltpu.sync_copy(ids_hbm.at[pl.ds(base, BPW)], idx_v)
    for j in range(BPW // 128):
        pltpu.async_copy(mem_hbm.at[idx_v.at[pl.ds(j * 128, 128)]],
                         rows_v.at[pl.ds(j * 128, 128)], sem)
    for j in range(BPW // 128):
        pltpu.make_async_copy(mem_hbm.at[idx_v.at[pl.ds(j * 128, 128)]],
                              rows_v.at[pl.ds(j * 128, 128)], sem).wait()
    pltpu.sync_copy(rows_v, cur_hbm.at[pl.ds(base, BPW)])


def _gru_body(msg_ref, cur_ref, wih_ref, whh_ref, bih_ref, bhh_ref,
              g_ref, bt_ref, out_ref):
    msg = msg_ref[...]
    cur = cur_ref[...]
    dn = (((1,), (1,)), ((), ()))
    gi = lax.dot_general(msg, wih_ref[...], dn,
                         preferred_element_type=jnp.float32) + bih_ref[...]
    gh = lax.dot_general(cur, whh_ref[...], dn,
                         preferred_element_type=jnp.float32) + bhh_ref[...]
    r = jax.nn.sigmoid(gi[:, :D] + gh[:, :D])
    z = jax.nn.sigmoid(gi[:, D:2 * D] + gh[:, D:2 * D])
    n = jnp.tanh(gi[:, 2 * D:] + r * gh[:, 2 * D:])
    h = (1.0 - z) * n + z * cur
    mu = jnp.mean(h, axis=-1, keepdims=True)
    var = jnp.mean((h - mu) ** 2, axis=-1, keepdims=True)
    out_ref[...] = (h - mu) * lax.rsqrt(var + 1e-5) * g_ref[...] + bt_ref[...]


_GRU_BLK = 2048


def _tc_gru(cur, msgs, W_ih, W_hh, b_ih, b_hh, g, bt):
    grid = B // _GRU_BLK
    return pl.pallas_call(
        _gru_body,
        grid=(grid,),
        in_specs=[
            pl.BlockSpec((_GRU_BLK, DMSG), lambda i: (i, 0)),
            pl.BlockSpec((_GRU_BLK, D), lambda i: (i, 0)),
            pl.BlockSpec((3 * D, DMSG), lambda i: (0, 0)),
            pl.BlockSpec((3 * D, D), lambda i: (0, 0)),
            pl.BlockSpec((1, 3 * D), lambda i: (0, 0)),
            pl.BlockSpec((1, 3 * D), lambda i: (0, 0)),
            pl.BlockSpec((1, D), lambda i: (0, 0)),
            pl.BlockSpec((1, D), lambda i: (0, 0)),
        ],
        out_specs=pl.BlockSpec((_GRU_BLK, D), lambda i: (i, 0)),
        out_shape=jax.ShapeDtypeStruct((B, D), jnp.float32),
    )(msgs, cur, W_ih, W_hh, b_ih, b_hh, g, bt)


_NCB = 3
_CPR = 128


def _emit_range_copy(src, dst, base, n_chunks, tail_rows, bufs, in_sems,
                     out_sems, compute_segment):
    def _in(k):
        off = base + k * _CPR
        return pltpu.make_async_copy(src.at[pl.ds(off, _CPR)],
                                     bufs.at[k % _NCB], in_sems.at[k % _NCB])

    def _out(k):
        off = base + k * _CPR
        return pltpu.make_async_copy(bufs.at[k % _NCB],
                                     dst.at[pl.ds(off, _CPR)],
                                     out_sems.at[k % _NCB])

    waited = set()
    for k in range(min(_NCB, n_chunks)):
        _in(k).start()
    for k in range(n_chunks):
        compute_segment(k, n_chunks)
        j = k - 2
        if j >= 0 and j + _NCB < n_chunks:
            _out(j).wait()
            waited.add(j)
            _in(j + _NCB).start()
        _in(k).wait()
        _out(k).start()
    for k in range(n_chunks):
        if k not in waited:
            _out(k).wait()
    if tail_rows:
        off = base + n_chunks * _CPR
        pltpu.sync_copy(src.at[pl.ds(off, tail_rows)],
                        bufs.at[0, pl.ds(0, tail_rows)])
        pltpu.sync_copy(bufs.at[0, pl.ds(0, tail_rows)],
                        dst.at[pl.ds(off, tail_rows)])


@functools.partial(
    pl.kernel,
    out_type=(jax.ShapeDtypeStruct((M, D), jnp.float32),
              jax.ShapeDtypeStruct((M,), jnp.float32)),
    mesh=plsc.VectorSubcoreMesh(**_MESH),
    scratch_types=[
        pltpu.VMEM((B,), jnp.int32),
        pltpu.VMEM((RNG,), jnp.int32),
        pltpu.VMEM((CAP,), jnp.int32),
        pltpu.VMEM((CAP,), jnp.int32),
        pltpu.VMEM((2, 128), jnp.int32),
        pltpu.VMEM((2, 128), jnp.int32),
        pltpu.VMEM((2, 128, D), jnp.float32),
        pltpu.VMEM((B,), jnp.float32),
        pltpu.VMEM((_NCB, _CPR, D), jnp.float32),
        pltpu.VMEM((RNG,), jnp.float32),
        pltpu.SemaphoreType.DMA((_NCB,)),
        pltpu.SemaphoreType.DMA((_NCB,)),
        pltpu.SemaphoreType.DMA((2,)),
        pltpu.SemaphoreType.DMA((2,)),
    ],
    compiler_params=pltpu.CompilerParams(needs_layout_passes=False),
)
def _sc_scatter(mem_hbm, lu_hbm, ids_hbm, ts_hbm, new_hbm,
                outm_hbm, outl_hbm,
                ids_v, tab_v, dstf_v, srcf_v, idxc_v, srcc_v, rows_v, tsall_v,
                cpb_v, lub_v, in_sems, out_sems, gr_sems, sr_sems):
    wid = _wid()
    base = wid * RNG
    lane = _lane_iota()

    pltpu.sync_copy(ids_hbm, ids_v)
    pltpu.sync_copy(ts_hbm, tsall_v)

    def _clear(t, carry):
        tab_v[pl.ds(t * 16, 16)] = jnp.full((16,), -1, jnp.int32)
        return carry
    lax.fori_loop(0, NTAB, _clear, 0)

    _UNR = 4

    def _scan(g, carry):
        comps = []
        for u in range(_UNR):
            v = g * _UNR + u
            ids16 = ids_v[pl.ds(v * 16, 16)]
            comps.append(plsc.sort_key_val(
                ids16 * 16384 + (v * 16 + lane),
                ids16 * 16384 + (v * 16 + lane))[0])
        for comp in comps:
            sid = lax.shift_right_logical(comp, 14)
            si = lax.bitwise_and(comp, 16383)
            nxt = _shift_up(sid)
            m = ((sid >= base) & (sid < base + RNG)
                 & ((sid != nxt) | (lane == 15)))
            plsc.store_scatter(tab_v, [sid - base], si, mask=m)
        return carry

    def _scan_segment(k, n_chunks):
        ngrp = NVREG // _UNR
        lo = k * ngrp // n_chunks
        hi = (k + 1) * ngrp // n_chunks
        lax.fori_loop(lo, hi, _scan, 0)

    @pl.when(wid < NW - 1)
    def _():
        _emit_range_copy(mem_hbm, outm_hbm, base, RNG // _CPR, RNG % _CPR,
                         cpb_v, in_sems, out_sems, _scan_segment)
        pltpu.sync_copy(lu_hbm.at[pl.ds(base, RNG)], lub_v)

    @pl.when(wid == NW - 1)
    def _():
        _emit_range_copy(mem_hbm, outm_hbm, base, RNG_LAST // _CPR,
                         RNG_LAST % _CPR, cpb_v, in_sems, out_sems,
                         _scan_segment)
        pltpu.sync_copy(lu_hbm.at[pl.ds(base, RNG_LAST)],
                        lub_v.at[pl.ds(0, RNG_LAST)])

    def _compact(t, off):
        tv = tab_v[pl.ds(t * 16, 16)]
        m = tv >= 0
        pc = plsc.cumsum(jnp.where(m, 1, 0).astype(jnp.int32))
        tgt = off + pc - 1
        plsc.store_scatter(dstf_v, [tgt], base + t * 16 + lane, mask=m)
        plsc.store_scatter(srcf_v, [tgt], tv, mask=m)
        return off + jnp.max(plsc.all_reduce_population_count(m))
    cnt = lax.fori_loop(0, NTAB, _compact, jnp.int32(0))

    nch = (cnt + 127) // 128

    @pl.when(cnt > 0)
    def _():
        pad_end = nch * 128

        def _pad(k, carry):
            pos = cnt + k * 16 + lane
            pm = pos < pad_end
            j = jnp.minimum(pos - cnt, cnt - 1)
            dv = plsc.load_gather(dstf_v, [j])
            sv = plsc.load_gather(srcf_v, [j])
            plsc.store_scatter(dstf_v, [pos], dv, mask=pm)
            plsc.store_scatter(srcf_v, [pos], sv, mask=pm)
            return carry
        lax.fori_loop(0, 8, _pad, 0)

    def _g_rows(s):
        return pltpu.make_async_copy(new_hbm.at[srcc_v.at[s]],
                                     rows_v.at[s], gr_sems.at[s])

    def _s_rows(s):
        return pltpu.make_async_copy(rows_v.at[s],
                                     outm_hbm.at[idxc_v.at[s]], sr_sems.at[s])

    def _pipe(c, carry):
        slot = lax.bitwise_and(c, 1)

        @pl.when(c < nch)
        def _():
            @pl.when(c >= 2)
            def _():
                _s_rows(slot).wait()

            def _fill(j, carry2):
                idxc_v[slot, pl.ds(j * 16, 16)] = (
                    dstf_v[pl.ds(c * 128 + j * 16, 16)])
                srcc_v[slot, pl.ds(j * 16, 16)] = (
                    srcf_v[pl.ds(c * 128 + j * 16, 16)])
                return carry2
            lax.fori_loop(0, 8, _fill, 0)
            _g_rows(slot).start()

        @pl.when(c >= 1)
        def _():
            pslot = lax.bitwise_and(c - 1, 1)
            _g_rows(pslot).wait()
            _s_rows(pslot).start()
        return carry
    lax.fori_loop(0, nch + 1, _pipe, 0)

    def _tsap(q, carry):
        pos = q * 16 + lane
        d16 = plsc.load_gather(dstf_v, [pos])
        s16 = plsc.load_gather(srcf_v, [pos])
        tsv = plsc.load_gather(tsall_v, [s16])
        plsc.store_scatter(lub_v, [d16 - base], tsv)
        return carry
    lax.fori_loop(0, nch * 8, _tsap, 0)

    @pl.when(wid < NW - 1)
    def _():
        pltpu.sync_copy(lub_v, outl_hbm.at[pl.ds(base, RNG)])

    @pl.when(wid == NW - 1)
    def _():
        pltpu.sync_copy(lub_v.at[pl.ds(0, RNG_LAST)],
                        outl_hbm.at[pl.ds(base, RNG_LAST)])

    @pl.when(nch >= 2)
    def _():
        _s_rows(lax.bitwise_and(nch, 1)).wait()

    @pl.when(nch >= 1)
    def _():
        _s_rows(lax.bitwise_and(nch - 1, 1)).wait()


def kernel(memory, last_update, unique_node_ids, unique_messages, timestamps,
           W_ih, W_hh, b_ih, b_hh, ln_gamma, ln_beta):
    ids = unique_node_ids.astype(jnp.int32)
    cur = _sc_gather(memory, ids)
    new_mem = _tc_gru(cur, unique_messages, W_ih, W_hh,
                      b_ih[None, :], b_hh[None, :],
                      ln_gamma[None, :], ln_beta[None, :])
    out_mem, out_lu = _sc_scatter(memory, last_update, ids, timestamps,
                                  new_mem)
    return out_mem, out_lu


# speedup vs baseline: 1.5204x; 1.0043x over previous
import functools

import jax
import jax.numpy as jnp
from jax import lax
from jax.experimental import pallas as pl
from jax.experimental.pallas import tpu as pltpu
from jax.experimental.pallas import tpu_sc as plsc

M = 100000
D = 128
DMSG = 256
B = 16384
NW = 32
BPW = B // NW
RNG = 3136
RNG_LAST = M - (NW - 1) * RNG
NVREG = B // 16
NTAB = RNG // 16
CAP = RNG + 64

_MESH = dict(core_axis_name="c", subcore_axis_name="s", num_cores=2,
             num_subcores=16)


def _wid():
    return lax.axis_index("s") * 2 + lax.axis_index("c")


def _lane_iota():
    return lax.iota(jnp.int32, 16)


def _shift_up(x):
    idx = jnp.minimum(_lane_iota() + 1, 15)
    dn = lax.GatherDimensionNumbers(
        offset_dims=(), collapsed_slice_dims=(0,), start_index_map=(0,))
    return lax.gather(x, idx[:, None], dn, (1,),
                      mode=lax.GatherScatterMode.PROMISE_IN_BOUNDS)


def _splat0(x):
    idx = jnp.zeros((16,), jnp.int32)
    dn = lax.GatherDimensionNumbers(
        offset_dims=(), collapsed_slice_dims=(0,), start_index_map=(0,))
    return lax.gather(x, idx[:, None], dn, (1,),
                      mode=lax.GatherScatterMode.PROMISE_IN_BOUNDS)


@functools.partial(
    pl.kernel,
    out_type=jax.ShapeDtypeStruct((B, D), jnp.float32),
    mesh=plsc.VectorSubcoreMesh(**_MESH),
    scratch_types=[
        pltpu.VMEM((BPW,), jnp.int32),
        pltpu.VMEM((BPW, D), jnp.float32),
        pltpu.SemaphoreType.DMA,
    ],
)
def _sc_gather(mem_hbm, ids_hbm, cur_hbm, idx_v, rows_v, sem):
    base = _wid() * BPW
    pltpu.sync_copy(ids_hbm.at[pl.ds(base, BPW)], idx_v)
    for j in range(BPW // 128):
        pltpu.async_copy(mem_hbm.at[idx_v.at[pl.ds(j * 128, 128)]],
                         rows_v.at[pl.ds(j * 128, 128)], sem)
    for j in range(BPW // 128):
        pltpu.make_async_copy(mem_hbm.at[idx_v.at[pl.ds(j * 128, 128)]],
                              rows_v.at[pl.ds(j * 128, 128)], sem).wait()
    pltpu.sync_copy(rows_v, cur_hbm.at[pl.ds(base, BPW)])


def _gru_body(msg_ref, cur_ref, wih_ref, whh_ref, bih_ref, bhh_ref,
              g_ref, bt_ref, out_ref):
    msg = msg_ref[...]
    cur = cur_ref[...]
    dn = (((1,), (1,)), ((), ()))
    gi = lax.dot_general(msg, wih_ref[...], dn,
                         preferred_element_type=jnp.float32) + bih_ref[...]
    gh = lax.dot_general(cur, whh_ref[...], dn,
                         preferred_element_type=jnp.float32) + bhh_ref[...]
    r = jax.nn.sigmoid(gi[:, :D] + gh[:, :D])
    z = jax.nn.sigmoid(gi[:, D:2 * D] + gh[:, D:2 * D])
    n = jnp.tanh(gi[:, 2 * D:] + r * gh[:, 2 * D:])
    h = (1.0 - z) * n + z * cur
    mu = jnp.mean(h, axis=-1, keepdims=True)
    var = jnp.mean((h - mu) ** 2, axis=-1, keepdims=True)
    out_ref[...] = (h - mu) * lax.rsqrt(var + 1e-5) * g_ref[...] + bt_ref[...]


_GRU_BLK = 4096


def _tc_gru(cur, msgs, W_ih, W_hh, b_ih, b_hh, g, bt):
    grid = B // _GRU_BLK
    return pl.pallas_call(
        _gru_body,
        grid=(grid,),
        in_specs=[
            pl.BlockSpec((_GRU_BLK, DMSG), lambda i: (i, 0)),
            pl.BlockSpec((_GRU_BLK, D), lambda i: (i, 0)),
            pl.BlockSpec((3 * D, DMSG), lambda i: (0, 0)),
            pl.BlockSpec((3 * D, D), lambda i: (0, 0)),
            pl.BlockSpec((1, 3 * D), lambda i: (0, 0)),
            pl.BlockSpec((1, 3 * D), lambda i: (0, 0)),
            pl.BlockSpec((1, D), lambda i: (0, 0)),
            pl.BlockSpec((1, D), lambda i: (0, 0)),
        ],
        out_specs=pl.BlockSpec((_GRU_BLK, D), lambda i: (i, 0)),
        out_shape=jax.ShapeDtypeStruct((B, D), jnp.float32),
    )(msgs, cur, W_ih, W_hh, b_ih, b_hh, g, bt)


_NCB = 3
_CPR = 128


def _emit_range_copy(src, dst, base, n_chunks, tail_rows, bufs, in_sems,
                     out_sems, compute_segment):
    def _in(k):
        off = base + k * _CPR
        return pltpu.make_async_copy(src.at[pl.ds(off, _CPR)],
                                     bufs.at[k % _NCB], in_sems.at[k % _NCB])

    def _out(k):
        off = base + k * _CPR
        return pltpu.make_async_copy(bufs.at[k % _NCB],
                                     dst.at[pl.ds(off, _CPR)],
                                     out_sems.at[k % _NCB])

    waited = set()
    for k in range(min(_NCB, n_chunks)):
        _in(k).start()
    for k in range(n_chunks):
        compute_segment(k, n_chunks)
        j = k - 2
        if j >= 0 and j + _NCB < n_chunks:
            _out(j).wait()
            waited.add(j)
            _in(j + _NCB).start()
        _in(k).wait()
        _out(k).start()
    for k in range(n_chunks):
        if k not in waited:
            _out(k).wait()
    if tail_rows:
        off = base + n_chunks * _CPR
        pltpu.sync_copy(src.at[pl.ds(off, tail_rows)],
                        bufs.at[0, pl.ds(0, tail_rows)])
        pltpu.sync_copy(bufs.at[0, pl.ds(0, tail_rows)],
                        dst.at[pl.ds(off, tail_rows)])


@functools.partial(
    pl.kernel,
    out_type=(jax.ShapeDtypeStruct((M, D), jnp.float32),
              jax.ShapeDtypeStruct((M,), jnp.float32)),
    mesh=plsc.VectorSubcoreMesh(**_MESH),
    scratch_types=[
        pltpu.VMEM((B,), jnp.int32),
        pltpu.VMEM((RNG,), jnp.int32),
        pltpu.VMEM((CAP,), jnp.int32),
        pltpu.VMEM((CAP,), jnp.int32),
        pltpu.VMEM((2, 128), jnp.int32),
        pltpu.VMEM((2, 128), jnp.int32),
        pltpu.VMEM((2, 128, D), jnp.float32),
        pltpu.VMEM((B,), jnp.float32),
        pltpu.VMEM((_NCB, _CPR, D), jnp.float32),
        pltpu.VMEM((RNG,), jnp.float32),
        pltpu.SemaphoreType.DMA((_NCB,)),
        pltpu.SemaphoreType.DMA((_NCB,)),
        pltpu.SemaphoreType.DMA((2,)),
        pltpu.SemaphoreType.DMA((2,)),
    ],
    compiler_params=pltpu.CompilerParams(needs_layout_passes=False),
)
def _sc_scatter(mem_hbm, lu_hbm, ids_hbm, ts_hbm, new_hbm,
                outm_hbm, outl_hbm,
                ids_v, tab_v, dstf_v, srcf_v, idxc_v, srcc_v, rows_v, tsall_v,
                cpb_v, lub_v, in_sems, out_sems, gr_sems, sr_sems):
    wid = _wid()
    base = wid * RNG
    lane = _lane_iota()

    pltpu.sync_copy(ids_hbm, ids_v)
    pltpu.sync_copy(ts_hbm, tsall_v)

    def _clear(t, carry):
        tab_v[pl.ds(t * 16, 16)] = jnp.full((16,), -1, jnp.int32)
        return carry
    lax.fori_loop(0, NTAB, _clear, 0)

    _UNR = 4

    def _scan(g, carry):
        comps = []
        for u in range(_UNR):
            v = g * _UNR + u
            ids16 = ids_v[pl.ds(v * 16, 16)]
            comps.append(plsc.sort_key_val(
                ids16 * 16384 + (v * 16 + lane),
                ids16 * 16384 + (v * 16 + lane))[0])
        for comp in comps:
            sid = lax.shift_right_logical(comp, 14)
            si = lax.bitwise_and(comp, 16383)
            nxt = _shift_up(sid)
            m = ((sid >= base) & (sid < base + RNG)
                 & ((sid != nxt) | (lane == 15)))
            plsc.store_scatter(tab_v, [sid - base], si, mask=m)
        return carry

    def _scan_segment(k, n_chunks):
        ngrp = NVREG // _UNR
        lo = k * ngrp // n_chunks
        hi = (k + 1) * ngrp // n_chunks
        lax.fori_loop(lo, hi, _scan, 0)

    @pl.when(wid < NW - 1)
    def _():
        _emit_range_copy(mem_hbm, outm_hbm, base, RNG // _CPR, RNG % _CPR,
                         cpb_v, in_sems, out_sems, _scan_segment)
        pltpu.sync_copy(lu_hbm.at[pl.ds(base, RNG)], lub_v)

    @pl.when(wid == NW - 1)
    def _():
        _emit_range_copy(mem_hbm, outm_hbm, base, RNG_LAST // _CPR,
                         RNG_LAST % _CPR, cpb_v, in_sems, out_sems,
                         _scan_segment)
        pltpu.sync_copy(lu_hbm.at[pl.ds(base, RNG_LAST)],
                        lub_v.at[pl.ds(0, RNG_LAST)])

    def _compact(t, off):
        tv = tab_v[pl.ds(t * 16, 16)]
        m = tv >= 0
        pc = plsc.cumsum(jnp.where(m, 1, 0).astype(jnp.int32))
        tgt = off + pc - 1
        plsc.store_scatter(dstf_v, [tgt], base + t * 16 + lane, mask=m)
        plsc.store_scatter(srcf_v, [tgt], tv, mask=m)
        return off + jnp.max(plsc.all_reduce_population_count(m))
    cnt = lax.fori_loop(0, NTAB, _compact, jnp.int32(0))

    nch = (cnt + 127) // 128

    @pl.when(cnt > 0)
    def _():
        pad_end = nch * 128

        def _pad(k, carry):
            pos = cnt + k * 16 + lane
            pm = pos < pad_end
            j = jnp.minimum(pos - cnt, cnt - 1)
            dv = plsc.load_gather(dstf_v, [j])
            sv = plsc.load_gather(srcf_v, [j])
            plsc.store_scatter(dstf_v, [pos], dv, mask=pm)
            plsc.store_scatter(srcf_v, [pos], sv, mask=pm)
            return carry
        lax.fori_loop(0, 8, _pad, 0)

    def _g_rows(s):
        return pltpu.make_async_copy(new_hbm.at[srcc_v.at[s]],
                                     rows_v.at[s], gr_sems.at[s])

    def _s_rows(s):
        return pltpu.make_async_copy(rows_v.at[s],
                                     outm_hbm.at[idxc_v.at[s]], sr_sems.at[s])

    def _pipe(c, carry):
        slot = lax.bitwise_and(c, 1)

        @pl.when(c < nch)
        def _():
            @pl.when(c >= 2)
            def _():
                _s_rows(slot).wait()

            def _fill(j, carry2):
                idxc_v[slot, pl.ds(j * 16, 16)] = (
                    dstf_v[pl.ds(c * 128 + j * 16, 16)])
                srcc_v[slot, pl.ds(j * 16, 16)] = (
                    srcf_v[pl.ds(c * 128 + j * 16, 16)])
                return carry2
            lax.fori_loop(0, 8, _fill, 0)
            _g_rows(slot).start()

        @pl.when(c >= 1)
        def _():
            pslot = lax.bitwise_and(c - 1, 1)
            _g_rows(pslot).wait()
            _s_rows(pslot).start()
        return carry
    lax.fori_loop(0, nch + 1, _pipe, 0)

    def _tsap(q, carry):
        pos = q * 16 + lane
        d16 = plsc.load_gather(dstf_v, [pos])
        s16 = plsc.load_gather(srcf_v, [pos])
        tsv = plsc.load_gather(tsall_v, [s16])
        plsc.store_scatter(lub_v, [d16 - base], tsv)
        return carry
    lax.fori_loop(0, nch * 8, _tsap, 0)

    @pl.when(wid < NW - 1)
    def _():
        pltpu.sync_copy(lub_v, outl_hbm.at[pl.ds(base, RNG)])

    @pl.when(wid == NW - 1)
    def _():
        pltpu.sync_copy(lub_v.at[pl.ds(0, RNG_LAST)],
                        outl_hbm.at[pl.ds(base, RNG_LAST)])

    @pl.when(nch >= 2)
    def _():
        _s_rows(lax.bitwise_and(nch, 1)).wait()

    @pl.when(nch >= 1)
    def _():
        _s_rows(lax.bitwise_and(nch - 1, 1)).wait()


def kernel(memory, last_update, unique_node_ids, unique_messages, timestamps,
           W_ih, W_hh, b_ih, b_hh, ln_gamma, ln_beta):
    ids = unique_node_ids.astype(jnp.int32)
    cur = _sc_gather(memory, ids)
    new_mem = _tc_gru(cur, unique_messages, W_ih, W_hh,
                      b_ih[None, :], b_hh[None, :],
                      ln_gamma[None, :], ln_beta[None, :])
    out_mem, out_lu = _sc_scatter(memory, last_update, ids, timestamps,
                                  new_mem)
    return out_mem, out_lu


# speedup vs baseline: 1.5487x; 1.0187x over previous
import functools

import jax
import jax.numpy as jnp
from jax import lax
from jax.experimental import pallas as pl
from jax.experimental.pallas import tpu as pltpu
from jax.experimental.pallas import tpu_sc as plsc

M = 100000
D = 128
DMSG = 256
B = 16384
NW = 32
BPW = B // NW
RNG = 3136
RNG_LAST = M - (NW - 1) * RNG
NVREG = B // 16
NTAB = RNG // 16
CAP = RNG + 64

_MESH = dict(core_axis_name="c", subcore_axis_name="s", num_cores=2,
             num_subcores=16)


def _wid():
    return lax.axis_index("s") * 2 + lax.axis_index("c")


def _lane_iota():
    return lax.iota(jnp.int32, 16)


def _shift_up(x):
    idx = jnp.minimum(_lane_iota() + 1, 15)
    dn = lax.GatherDimensionNumbers(
        offset_dims=(), collapsed_slice_dims=(0,), start_index_map=(0,))
    return lax.gather(x, idx[:, None], dn, (1,),
                      mode=lax.GatherScatterMode.PROMISE_IN_BOUNDS)


def _splat0(x):
    idx = jnp.zeros((16,), jnp.int32)
    dn = lax.GatherDimensionNumbers(
        offset_dims=(), collapsed_slice_dims=(0,), start_index_map=(0,))
    return lax.gather(x, idx[:, None], dn, (1,),
                      mode=lax.GatherScatterMode.PROMISE_IN_BOUNDS)


@functools.partial(
    pl.kernel,
    out_type=jax.ShapeDtypeStruct((B, D), jnp.float32),
    mesh=plsc.VectorSubcoreMesh(**_MESH),
    scratch_types=[
        pltpu.VMEM((BPW,), jnp.int32),
        pltpu.VMEM((BPW, D), jnp.float32),
        pltpu.SemaphoreType.DMA,
    ],
)
def _sc_gather(mem_hbm, ids_hbm, cur_hbm, idx_v, rows_v, sem):
    base = _wid() * BPW
    pltpu.sync_copy(ids_hbm.at[pl.ds(base, BPW)], idx_v)
    for j in range(BPW // 128):
        pltpu.async_copy(mem_hbm.at[idx_v.at[pl.ds(j * 128, 128)]],
                         rows_v.at[pl.ds(j * 128, 128)], sem)
    for j in range(BPW // 128):
        pltpu.make_async_copy(mem_hbm.at[idx_v.at[pl.ds(j * 128, 128)]],
                              rows_v.at[pl.ds(j * 128, 128)], sem).wait()
    pltpu.sync_copy(rows_v, cur_hbm.at[pl.ds(base, BPW)])


def _gru_body(msg_ref, cur_ref, wih_ref, whh_ref, bih_ref, bhh_ref,
              g_ref, bt_ref, out_ref):
    msg = msg_ref[...]
    cur = cur_ref[...]
    dn = (((1,), (1,)), ((), ()))
    gi = lax.dot_general(msg, wih_ref[...], dn,
                         preferred_element_type=jnp.float32) + bih_ref[...]
    gh = lax.dot_general(cur, whh_ref[...], dn,
                         preferred_element_type=jnp.float32) + bhh_ref[...]
    r = jax.nn.sigmoid(gi[:, :D] + gh[:, :D])
    z = jax.nn.sigmoid(gi[:, D:2 * D] + gh[:, D:2 * D])
    n = jnp.tanh(gi[:, 2 * D:] + r * gh[:, 2 * D:])
    h = (1.0 - z) * n + z * cur
    mu = jnp.mean(h, axis=-1, keepdims=True)
    var = jnp.mean((h - mu) ** 2, axis=-1, keepdims=True)
    out_ref[...] = (h - mu) * lax.rsqrt(var + 1e-5) * g_ref[...] + bt_ref[...]


_GRU_BLK = 4096


def _tc_gru(cur, msgs, W_ih, W_hh, b_ih, b_hh, g, bt):
    grid = B // _GRU_BLK
    return pl.pallas_call(
        _gru_body,
        grid=(grid,),
        in_specs=[
            pl.BlockSpec((_GRU_BLK, DMSG), lambda i: (i, 0)),
            pl.BlockSpec((_GRU_BLK, D), lambda i: (i, 0)),
            pl.BlockSpec((3 * D, DMSG), lambda i: (0, 0)),
            pl.BlockSpec((3 * D, D), lambda i: (0, 0)),
            pl.BlockSpec((1, 3 * D), lambda i: (0, 0)),
            pl.BlockSpec((1, 3 * D), lambda i: (0, 0)),
            pl.BlockSpec((1, D), lambda i: (0, 0)),
            pl.BlockSpec((1, D), lambda i: (0, 0)),
        ],
        out_specs=pl.BlockSpec((_GRU_BLK, D), lambda i: (i, 0)),
        out_shape=jax.ShapeDtypeStruct((B, D), jnp.float32),
    )(msgs, cur, W_ih, W_hh, b_ih, b_hh, g, bt)


_NCB = 2
_CPR = 192


def _emit_range_copy(src, dst, base, n_chunks, tail_rows, bufs, in_sems,
                     out_sems, compute_segment):
    def _in(k):
        off = base + k * _CPR
        return pltpu.make_async_copy(src.at[pl.ds(off, _CPR)],
                                     bufs.at[k % _NCB], in_sems.at[k % _NCB])

    def _out(k):
        off = base + k * _CPR
        return pltpu.make_async_copy(bufs.at[k % _NCB],
                                     dst.at[pl.ds(off, _CPR)],
                                     out_sems.at[k % _NCB])

    waited = set()
    for k in range(min(_NCB, n_chunks)):
        _in(k).start()
    for k in range(n_chunks):
        compute_segment(k, n_chunks)
        j = k + 1 - _NCB
        if j >= 0 and j + _NCB < n_chunks:
            _out(j).wait()
            waited.add(j)
            _in(j + _NCB).start()
        _in(k).wait()
        _out(k).start()
    for k in range(n_chunks):
        if k not in waited:
            _out(k).wait()
    if tail_rows:
        off = base + n_chunks * _CPR
        pltpu.sync_copy(src.at[pl.ds(off, tail_rows)],
                        bufs.at[0, pl.ds(0, tail_rows)])
        pltpu.sync_copy(bufs.at[0, pl.ds(0, tail_rows)],
                        dst.at[pl.ds(off, tail_rows)])


@functools.partial(
    pl.kernel,
    out_type=(jax.ShapeDtypeStruct((M, D), jnp.float32),
              jax.ShapeDtypeStruct((M,), jnp.float32)),
    mesh=plsc.VectorSubcoreMesh(**_MESH),
    scratch_types=[
        pltpu.VMEM((B,), jnp.int32),
        pltpu.VMEM((RNG,), jnp.int32),
        pltpu.VMEM((CAP,), jnp.int32),
        pltpu.VMEM((CAP,), jnp.int32),
        pltpu.VMEM((2, 128), jnp.int32),
        pltpu.VMEM((2, 128), jnp.int32),
        pltpu.VMEM((2, 128, D), jnp.float32),
        pltpu.VMEM((B,), jnp.float32),
        pltpu.VMEM((_NCB, _CPR, D), jnp.float32),
        pltpu.VMEM((RNG,), jnp.float32),
        pltpu.SemaphoreType.DMA((_NCB,)),
        pltpu.SemaphoreType.DMA((_NCB,)),
        pltpu.SemaphoreType.DMA((2,)),
        pltpu.SemaphoreType.DMA((2,)),
    ],
    compiler_params=pltpu.CompilerParams(needs_layout_passes=False),
)
def _sc_scatter(mem_hbm, lu_hbm, ids_hbm, ts_hbm, new_hbm,
                outm_hbm, outl_hbm,
                ids_v, tab_v, dstf_v, srcf_v, idxc_v, srcc_v, rows_v, tsall_v,
                cpb_v, lub_v, in_sems, out_sems, gr_sems, sr_sems):
    wid = _wid()
    base = wid * RNG
    lane = _lane_iota()

    pltpu.sync_copy(ids_hbm, ids_v)
    pltpu.sync_copy(ts_hbm, tsall_v)

    def _clear(t, carry):
        tab_v[pl.ds(t * 16, 16)] = jnp.full((16,), -1, jnp.int32)
        return carry
    lax.fori_loop(0, NTAB, _clear, 0)

    _UNR = 4

    def _scan(g, carry):
        comps = []
        for u in range(_UNR):
            v = g * _UNR + u
            ids16 = ids_v[pl.ds(v * 16, 16)]
            comps.append(plsc.sort_key_val(
                ids16 * 16384 + (v * 16 + lane),
                ids16 * 16384 + (v * 16 + lane))[0])
        for comp in comps:
            sid = lax.shift_right_logical(comp, 14)
            si = lax.bitwise_and(comp, 16383)
            nxt = _shift_up(sid)
            m = ((sid >= base) & (sid < base + RNG)
                 & ((sid != nxt) | (lane == 15)))
            plsc.store_scatter(tab_v, [sid - base], si, mask=m)
        return carry

    def _scan_segment(k, n_chunks):
        ngrp = NVREG // _UNR
        lo = k * ngrp // n_chunks
        hi = (k + 1) * ngrp // n_chunks
        lax.fori_loop(lo, hi, _scan, 0)

    @pl.when(wid < NW - 1)
    def _():
        _emit_range_copy(mem_hbm, outm_hbm, base, RNG // _CPR, RNG % _CPR,
                         cpb_v, in_sems, out_sems, _scan_segment)
        pltpu.sync_copy(lu_hbm.at[pl.ds(base, RNG)], lub_v)

    @pl.when(wid == NW - 1)
    def _():
        _emit_range_copy(mem_hbm, outm_hbm, base, RNG_LAST // _CPR,
                         RNG_LAST % _CPR, cpb_v, in_sems, out_sems,
                         _scan_segment)
        pltpu.sync_copy(lu_hbm.at[pl.ds(base, RNG_LAST)],
                        lub_v.at[pl.ds(0, RNG_LAST)])

    def _compact(t, off):
        tv = tab_v[pl.ds(t * 16, 16)]
        m = tv >= 0
        pc = plsc.cumsum(jnp.where(m, 1, 0).astype(jnp.int32))
        tgt = off + pc - 1
        plsc.store_scatter(dstf_v, [tgt], base + t * 16 + lane, mask=m)
        plsc.store_scatter(srcf_v, [tgt], tv, mask=m)
        return off + jnp.max(plsc.all_reduce_population_count(m))
    cnt = lax.fori_loop(0, NTAB, _compact, jnp.int32(0))

    nch = (cnt + 127) // 128

    @pl.when(cnt > 0)
    def _():
        pad_end = nch * 128

        def _pad(k, carry):
            pos = cnt + k * 16 + lane
            pm = pos < pad_end
            j = jnp.minimum(pos - cnt, cnt - 1)
            dv = plsc.load_gather(dstf_v, [j])
            sv = plsc.load_gather(srcf_v, [j])
            plsc.store_scatter(dstf_v, [pos], dv, mask=pm)
            plsc.store_scatter(srcf_v, [pos], sv, mask=pm)
            return carry
        lax.fori_loop(0, 8, _pad, 0)

    def _g_rows(s):
        return pltpu.make_async_copy(new_hbm.at[srcc_v.at[s]],
                                     rows_v.at[s], gr_sems.at[s])

    def _s_rows(s):
        return pltpu.make_async_copy(rows_v.at[s],
                                     outm_hbm.at[idxc_v.at[s]], sr_sems.at[s])

    def _pipe(c, carry):
        slot = lax.bitwise_and(c, 1)

        @pl.when(c < nch)
        def _():
            @pl.when(c >= 2)
            def _():
                _s_rows(slot).wait()

            def _fill(j, carry2):
                idxc_v[slot, pl.ds(j * 16, 16)] = (
                    dstf_v[pl.ds(c * 128 + j * 16, 16)])
                srcc_v[slot, pl.ds(j * 16, 16)] = (
                    srcf_v[pl.ds(c * 128 + j * 16, 16)])
                return carry2
            lax.fori_loop(0, 8, _fill, 0)
            _g_rows(slot).start()

        @pl.when(c >= 1)
        def _():
            pslot = lax.bitwise_and(c - 1, 1)
            _g_rows(pslot).wait()
            _s_rows(pslot).start()
        return carry
    lax.fori_loop(0, nch + 1, _pipe, 0)

    def _tsap(q, carry):
        pos = q * 16 + lane
        d16 = plsc.load_gather(dstf_v, [pos])
        s16 = plsc.load_gather(srcf_v, [pos])
        tsv = plsc.load_gather(tsall_v, [s16])
        plsc.store_scatter(lub_v, [d16 - base], tsv)
        return carry
    lax.fori_loop(0, nch * 8, _tsap, 0)

    @pl.when(wid < NW - 1)
    def _():
        pltpu.sync_copy(lub_v, outl_hbm.at[pl.ds(base, RNG)])

    @pl.when(wid == NW - 1)
    def _():
        pltpu.sync_copy(lub_v.at[pl.ds(0, RNG_LAST)],
                        outl_hbm.at[pl.ds(base, RNG_LAST)])

    @pl.when(nch >= 2)
    def _():
        _s_rows(lax.bitwise_and(nch, 1)).wait()

    @pl.when(nch >= 1)
    def _():
        _s_rows(lax.bitwise_and(nch - 1, 1)).wait()


def kernel(memory, last_update, unique_node_ids, unique_messages, timestamps,
           W_ih, W_hh, b_ih, b_hh, ln_gamma, ln_beta):
    ids = unique_node_ids.astype(jnp.int32)
    cur = _sc_gather(memory, ids)
    new_mem = _tc_gru(cur, unique_messages, W_ih, W_hh,
                      b_ih[None, :], b_hh[None, :],
                      ln_gamma[None, :], ln_beta[None, :])
    out_mem, out_lu = _sc_scatter(memory, last_update, ids, timestamps,
                                  new_mem)
    return out_mem, out_lu
